# baseline (speedup 1.0000x reference)
import functools

import jax
import jax.numpy as jnp
from jax import lax
from jax.experimental import pallas as pl
from jax.experimental.pallas import tpu as pltpu
from jax.experimental.pallas import tpu_sc as plsc

B = 16384
F = 26
V = 100000
D = 50
DP = 128
NUM = 13

NC = 2
NS = 16
NW = NC * NS
TOTAL = B * F
PER_W = TOTAL // NW
CHUNK = 512
G = CHUNK // 128
N_CHUNKS = PER_W // CHUNK


@functools.cache
def _make_sc_gather():
    mesh = plsc.VectorSubcoreMesh(
        core_axis_name="c", subcore_axis_name="s", num_cores=NC, num_subcores=NS
    )

    @functools.partial(
        pl.kernel,
        mesh=mesh,
        out_type=jax.ShapeDtypeStruct((TOTAL, DP), jnp.float32),
        scratch_types=[
            pltpu.VMEM((G, 128), jnp.int32),
            pltpu.VMEM((CHUNK, DP), jnp.float32),
            pltpu.SemaphoreType.DMA,
        ],
        compiler_params=pltpu.CompilerParams(use_tc_tiling_on_sc=False),
    )
    def _sc_gather(table_hbm, idx_hbm, out_hbm, idx_v, rows_v, sem):
        wid = lax.axis_index("s") * NC + lax.axis_index("c")
        base = wid * (PER_W // CHUNK)

        def body(i, _):
            row = base + i
            pltpu.sync_copy(idx_hbm.at[row], idx_v)
            copies = [
                pltpu.async_copy(
                    table_hbm.at[idx_v.at[j]],
                    rows_v.at[pl.ds(j * 128, 128)], sem)
                for j in range(G)
            ]
            for c in copies:
                c.wait()
            pltpu.sync_copy(rows_v, out_hbm.at[pl.ds(row * CHUNK, CHUNK)])
            return ()

        lax.fori_loop(0, N_CHUNKS, body, ())

    return _sc_gather


TB = 12800
NVB = -(-V // TB)
TAIL = V - (NVB - 1) * TB


def _tr_body(t_ref, out_hbm, scr_out, sem_out):
    f = pl.program_id(0)
    v = pl.program_id(1)
    x = t_ref[0]
    eye = (jax.lax.broadcasted_iota(jnp.int32, (D, D), 0)
           == jax.lax.broadcasted_iota(jnp.int32, (D, D), 1))
    xt = lax.dot_general(x, eye.astype(jnp.float32),
                         (((0,), (0,)), ((), ())),
                         preferred_element_type=jnp.float32,
                         precision=lax.Precision.HIGHEST)
    scr_out[...] = jnp.pad(xt, ((0, 0), (0, DP - D)))
    base = f * V + v * TB

    @pl.when(v < NVB - 1)
    def _full():
        cp = pltpu.make_async_copy(
            scr_out, out_hbm.at[pl.ds(base, TB)], sem_out)
        cp.start()
        cp.wait()

    @pl.when(v == NVB - 1)
    def _tail():
        cp = pltpu.make_async_copy(
            scr_out.at[pl.ds(0, TAIL)], out_hbm.at[pl.ds(base, TAIL)],
            sem_out)
        cp.start()
        cp.wait()


def _transpose_pad(tables_t):
    grid = (F, NVB)
    return pl.pallas_call(
        _tr_body,
        grid=grid,
        in_specs=[pl.BlockSpec((1, D, TB), lambda f, v: (f, 0, v))],
        out_specs=pl.BlockSpec(memory_space=pltpu.HBM),
        out_shape=jax.ShapeDtypeStruct((F * V, DP), jnp.float32),
        scratch_shapes=[
            pltpu.VMEM((TB, DP), jnp.float32),
            pltpu.SemaphoreType.DMA,
        ],
    )(tables_t)


BB = 1024


def _mlp_body(emb_ref, xnum_ref, w1a_ref, w1b_ref, b1_ref, w2_ref, b2_ref,
              w3_ref, b3_ref, w4_ref, b4_ref, out_ref):
    h = jnp.dot(emb_ref[...], w1a_ref[...], preferred_element_type=jnp.float32,
                precision=lax.Precision.HIGHEST)
    h = h + jnp.dot(xnum_ref[...], w1b_ref[...],
                    preferred_element_type=jnp.float32,
                    precision=lax.Precision.HIGHEST)
    h = jnp.maximum(h + b1_ref[...], 0.0)
    h = jnp.maximum(
        jnp.dot(h, w2_ref[...], preferred_element_type=jnp.float32,
                precision=lax.Precision.HIGHEST)
        + b2_ref[...], 0.0)
    h = jnp.maximum(
        jnp.dot(h, w3_ref[...], preferred_element_type=jnp.float32,
                precision=lax.Precision.HIGHEST)
        + b3_ref[...], 0.0)
    out_ref[...] = (
        jnp.dot(h, w4_ref[...], preferred_element_type=jnp.float32,
                precision=lax.Precision.HIGHEST)
        + b4_ref[...])


def _mlp(emb, x_num, W1a, W1b, b1, W2, b2, W3, b3, W4, b4):
    grid = (B // BB,)
    full = lambda shape: pl.BlockSpec(shape, lambda i: (0, 0))
    return pl.pallas_call(
        _mlp_body,
        grid=grid,
        in_specs=[
            pl.BlockSpec((BB, F * DP), lambda i: (i, 0)),
            pl.BlockSpec((BB, NUM), lambda i: (i, 0)),
            full((F * DP, 512)),
            full((NUM, 512)),
            full((1, 512)),
            full((512, 256)),
            full((1, 256)),
            full((256, 32)),
            full((1, 32)),
            full((32, 1)),
            full((1, 1)),
        ],
        out_specs=pl.BlockSpec((BB, 1), lambda i: (i, 0)),
        out_shape=jax.ShapeDtypeStruct((B, 1), jnp.float32),
    )(emb, x_num, W1a, W1b, b1, W2, b2, W3, b3, W4, b4)


def kernel(x_num, x_cat, tables, W1, b1, W2, b2, W3, b3, W4, b4):
    idx = (x_cat + jnp.arange(F, dtype=x_cat.dtype) * V).reshape(
        TOTAL // CHUNK, G, 128)
    table_pad = _transpose_pad(jnp.transpose(tables, (0, 2, 1)))
    emb = _make_sc_gather()(table_pad, idx).reshape(B, F * DP)
    W1a = jnp.pad(W1[: F * D].reshape(F, D, 512),
                  ((0, 0), (0, DP - D), (0, 0))).reshape(F * DP, 512)
    W1b = W1[F * D:]
    return _mlp(emb, x_num, W1a, W1b, b1.reshape(1, -1), W2, b2.reshape(1, -1),
                W3, b3.reshape(1, -1), W4, b4.reshape(1, -1))


# speedup vs baseline: 1.2688x; 1.2688x over previous
import functools

import jax
import jax.numpy as jnp
from jax import lax
from jax.experimental import pallas as pl
from jax.experimental.pallas import tpu as pltpu
from jax.experimental.pallas import tpu_sc as plsc

B = 16384
F = 26
V = 100000
D = 50
DP = 128
NUM = 13

NC = 2
NS = 16
NW = NC * NS
TOTAL = B * F
PER_W = TOTAL // NW
CHUNK = 512
G = CHUNK // 128
N_CHUNKS = PER_W // CHUNK


@functools.cache
def _make_sc_gather():
    mesh = plsc.VectorSubcoreMesh(
        core_axis_name="c", subcore_axis_name="s", num_cores=NC, num_subcores=NS
    )

    @functools.partial(
        pl.kernel,
        mesh=mesh,
        out_type=jax.ShapeDtypeStruct((TOTAL, DP), jnp.float32),
        scratch_types=[
            pltpu.VMEM((G, 128), jnp.int32),
            pltpu.VMEM((CHUNK, DP), jnp.float32),
            pltpu.SemaphoreType.DMA,
        ],
        compiler_params=pltpu.CompilerParams(use_tc_tiling_on_sc=False),
    )
    def _sc_gather(table_hbm, idx_hbm, out_hbm, idx_v, rows_v, sem):
        wid = lax.axis_index("s") * NC + lax.axis_index("c")
        base = wid * (PER_W // CHUNK)

        def body(i, _):
            row = base + i
            pltpu.sync_copy(idx_hbm.at[row], idx_v)
            copies = [
                pltpu.async_copy(
                    table_hbm.at[idx_v.at[j]],
                    rows_v.at[pl.ds(j * 128, 128)], sem)
                for j in range(G)
            ]
            for c in copies:
                c.wait()
            pltpu.sync_copy(rows_v, out_hbm.at[pl.ds(row * CHUNK, CHUNK)])
            return ()

        lax.fori_loop(0, N_CHUNKS, body, ())

    return _sc_gather


TB = 12800
NVB = -(-V // TB)
TAIL = V - (NVB - 1) * TB


def _tr_body(t_ref, out_hbm, scr_out, sem_out):
    f = pl.program_id(0)
    v = pl.program_id(1)
    s = f * NVB + v
    slot = lax.rem(s, 2)

    @pl.when(jnp.logical_and(s >= 2, v != 1))
    def _():
        pltpu.make_async_copy(
            scr_out.at[slot], out_hbm.at[pl.ds(0, TB)], sem_out).wait()

    @pl.when(jnp.logical_and(s >= 2, v == 1))
    def _():
        pltpu.make_async_copy(
            scr_out.at[slot, pl.ds(0, TAIL)], out_hbm.at[pl.ds(0, TAIL)],
            sem_out).wait()

    x = t_ref[0]
    eye = (jax.lax.broadcasted_iota(jnp.int32, (D, D), 0)
           == jax.lax.broadcasted_iota(jnp.int32, (D, D), 1))
    xt = lax.dot_general(x, eye.astype(jnp.float32),
                         (((0,), (0,)), ((), ())),
                         preferred_element_type=jnp.float32,
                         precision=lax.Precision.HIGHEST)
    scr_out[slot] = jnp.pad(xt, ((0, 0), (0, DP - D)))
    base = f * V + v * TB

    @pl.when(v < NVB - 1)
    def _full():
        pltpu.make_async_copy(
            scr_out.at[slot], out_hbm.at[pl.ds(base, TB)], sem_out).start()

    @pl.when(v == NVB - 1)
    def _tail():
        pltpu.make_async_copy(
            scr_out.at[slot, pl.ds(0, TAIL)], out_hbm.at[pl.ds(base, TAIL)],
            sem_out).start()

    @pl.when(s == F * NVB - 1)
    def _drain():
        pltpu.make_async_copy(
            scr_out.at[1 - slot], out_hbm.at[pl.ds(0, TB)], sem_out).wait()
        pltpu.make_async_copy(
            scr_out.at[slot, pl.ds(0, TAIL)], out_hbm.at[pl.ds(0, TAIL)],
            sem_out).wait()


def _transpose_pad(tables_t):
    grid = (F, NVB)
    return pl.pallas_call(
        _tr_body,
        grid=grid,
        in_specs=[pl.BlockSpec((1, D, TB), lambda f, v: (f, 0, v))],
        out_specs=pl.BlockSpec(memory_space=pltpu.HBM),
        out_shape=jax.ShapeDtypeStruct((F * V, DP), jnp.float32),
        scratch_shapes=[
            pltpu.VMEM((2, TB, DP), jnp.float32),
            pltpu.SemaphoreType.DMA,
        ],
    )(tables_t)


BB = 1024


def _mlp_body(emb_ref, xnum_ref, w1a_ref, w1b_ref, b1_ref, w2_ref, b2_ref,
              w3_ref, b3_ref, w4_ref, b4_ref, out_ref):
    h = jnp.dot(emb_ref[...], w1a_ref[...], preferred_element_type=jnp.float32,
                precision=lax.Precision.HIGHEST)
    h = h + jnp.dot(xnum_ref[...], w1b_ref[...],
                    preferred_element_type=jnp.float32,
                    precision=lax.Precision.HIGHEST)
    h = jnp.maximum(h + b1_ref[...], 0.0)
    h = jnp.maximum(
        jnp.dot(h, w2_ref[...], preferred_element_type=jnp.float32,
                precision=lax.Precision.HIGHEST)
        + b2_ref[...], 0.0)
    h = jnp.maximum(
        jnp.dot(h, w3_ref[...], preferred_element_type=jnp.float32,
                precision=lax.Precision.HIGHEST)
        + b3_ref[...], 0.0)
    out_ref[...] = (
        jnp.dot(h, w4_ref[...], preferred_element_type=jnp.float32,
                precision=lax.Precision.HIGHEST)
        + b4_ref[...])


def _mlp(emb, x_num, W1a, W1b, b1, W2, b2, W3, b3, W4, b4):
    grid = (B // BB,)
    full = lambda shape: pl.BlockSpec(shape, lambda i: (0, 0))
    return pl.pallas_call(
        _mlp_body,
        grid=grid,
        in_specs=[
            pl.BlockSpec((BB, F * DP), lambda i: (i, 0)),
            pl.BlockSpec((BB, NUM), lambda i: (i, 0)),
            full((F * DP, 512)),
            full((NUM, 512)),
            full((1, 512)),
            full((512, 256)),
            full((1, 256)),
            full((256, 32)),
            full((1, 32)),
            full((32, 1)),
            full((1, 1)),
        ],
        out_specs=pl.BlockSpec((BB, 1), lambda i: (i, 0)),
        out_shape=jax.ShapeDtypeStruct((B, 1), jnp.float32),
    )(emb, x_num, W1a, W1b, b1, W2, b2, W3, b3, W4, b4)


def kernel(x_num, x_cat, tables, W1, b1, W2, b2, W3, b3, W4, b4):
    idx = (x_cat + jnp.arange(F, dtype=x_cat.dtype) * V).reshape(
        TOTAL // CHUNK, G, 128)
    table_pad = _transpose_pad(jnp.transpose(tables, (0, 2, 1)))
    emb = _make_sc_gather()(table_pad, idx).reshape(B, F * DP)
    W1a = jnp.pad(W1[: F * D].reshape(F, D, 512),
                  ((0, 0), (0, DP - D), (0, 0))).reshape(F * DP, 512)
    W1b = W1[F * D:]
    return _mlp(emb, x_num, W1a, W1b, b1.reshape(1, -1), W2, b2.reshape(1, -1),
                W3, b3.reshape(1, -1), W4, b4.reshape(1, -1))
